# baseline (speedup 1.0000x reference)
import functools

import jax
import jax.numpy as jnp
from jax import lax
from jax.experimental import pallas as pl
from jax.experimental.pallas import tpu as pltpu
from jax.experimental.pallas import tpu_sc as plsc

_B = 4096
_M = 10
_N = 1000
_NC = 2
_NS = 16
_NW = _NC * _NS
_L = 16
_BPW = _B // _NW
_CH = 16
_NCH = _BPW // _CH


def _tables_body(p1ab_ref, p2ab_ref, p1ba_ref, p2ba_ref, cn_ref, nrm_ref):
    for k, (p1, p2) in enumerate(((p1ab_ref, p2ab_ref), (p1ba_ref, p2ba_ref))):
        t = p2[:, :]
        m = jnp.max(t, axis=0)
        s = jnp.sum(jnp.exp(t - m[None, :]), axis=0)
        cn_ref[k, :] = jnp.log(s) + m
        v = p1[:]
        mv = jnp.max(v)
        nrm_ref[k] = jnp.log(jnp.sum(jnp.exp(v - mv))) + mv


def _sc_body(samples_hbm, p1ab_hbm, cnab_hbm, p1ba_hbm, cnba_hbm,
             p2ab_hbm, p2ba_hbm, out_hbm,
             buf0, buf1, idx_a, idx_b, fidx, gv, g1, g2, out_v, sem0, sem1):
    wid = lax.axis_index("s") * _NC + lax.axis_index("c")
    base = wid * _BPW

    bufs = (buf0, buf1)
    sems = (sem0, sem1)

    def _start_chunk(c, slot):
        return [pltpu.async_copy(
            samples_hbm.at[pl.ds((base + c * _CH + l) * (_M * _N), 2 * _N)],
            bufs[slot].at[pl.ds(l * 2 * _N, 2 * _N)], sems[slot])
            for l in range(_CH)]

    handles = [None, None]
    handles[0] = _start_chunk(0, 0)

    row = lax.broadcasted_iota(jnp.int32, (_L,), 0)
    lanebase = row * (2 * _N)
    ninf = jnp.full((_L,), -jnp.inf, jnp.float32)

    for c in range(_NCH):
        if c + 1 < _NCH:
            handles[(c + 1) % 2] = _start_chunk(c + 1, (c + 1) % 2)
        for h in handles[c % 2]:
            h.wait()
        buf = bufs[c % 2]

        def body(p, carry, buf=buf):
            va, ia, vb, ib, i0, i1 = carry
            x0 = plsc.load_gather(buf, [i0])
            x1 = plsc.load_gather(buf, [i1])
            u0 = x0 > va
            u1 = x1 > vb
            va = jnp.where(u0, x0, va)
            ia = jnp.where(u0, i0, ia)
            vb = jnp.where(u1, x1, vb)
            ib = jnp.where(u1, i1, ib)
            return va, ia, vb, ib, i0 + 1, i1 + 1

        va, ia, vb, ib, _, _ = lax.fori_loop(
            0, _N, body, (ninf, lanebase, ninf, lanebase, lanebase,
                          lanebase + _N))
        idx_a[pl.ds(c * _L, _L)] = ia - lanebase
        idx_b[pl.ds(c * _L, _L)] = ib - lanebase - _N

    for model, (idx1, p1_hbm, cn_hbm, p2_hbm) in enumerate((
            (idx_a, p1ab_hbm, cnab_hbm, p2ab_hbm),
            (idx_b, p1ba_hbm, cnba_hbm, p2ba_hbm))):
        for c in range(_NCH):
            va = idx_a[pl.ds(c * _L, _L)]
            vb = idx_b[pl.ds(c * _L, _L)]
            f = vb * _N + va if model == 0 else va * _N + vb
            fidx[pl.ds(c * _L, _L)] = f
        h2 = pltpu.async_copy(p2_hbm.at[fidx], gv, sem0)
        hp = pltpu.async_copy(p1_hbm.at[idx1], g1, sem0)
        hc = pltpu.async_copy(cn_hbm.at[idx1], g2, sem0)
        h2.wait()
        hp.wait()
        hc.wait()
        acc = jnp.zeros((_L,), jnp.float32)
        for c in range(_NCH):
            s = pl.ds(c * _L, _L)
            acc = acc + gv[s] + g1[s] - g2[s]
        out_v[model, :] = acc

    pltpu.sync_copy(out_v, out_hbm.at[wid])


@functools.cache
def _make_sc_kernel():
    mesh = plsc.VectorSubcoreMesh(core_axis_name="c", subcore_axis_name="s",
                                  num_cores=_NC, num_subcores=_NS)
    return pl.kernel(
        _sc_body,
        mesh=mesh,
        compiler_params=pltpu.CompilerParams(needs_layout_passes=False),
        out_type=jax.ShapeDtypeStruct((_NW, 2, _L), jnp.float32),
        scratch_types=[
            pltpu.VMEM((_CH * 2 * _N,), jnp.float32),
            pltpu.VMEM((_CH * 2 * _N,), jnp.float32),
            pltpu.VMEM((_BPW,), jnp.int32),
            pltpu.VMEM((_BPW,), jnp.int32),
            pltpu.VMEM((_BPW,), jnp.int32),
            pltpu.VMEM((_BPW,), jnp.float32),
            pltpu.VMEM((_BPW,), jnp.float32),
            pltpu.VMEM((_BPW,), jnp.float32),
            pltpu.VMEM((2, _L), jnp.float32),
            pltpu.SemaphoreType.DMA,
            pltpu.SemaphoreType.DMA,
        ],
    )


def kernel(samples, P_1_AB, P_2_1_AB, P_1_BA, P_2_1_BA, gamma):
    B, M, N = samples.shape

    cn, nrm = pl.pallas_call(
        _tables_body,
        in_specs=[
            pl.BlockSpec((N,), lambda: (0,)),
            pl.BlockSpec((N, N), lambda: (0, 0)),
            pl.BlockSpec((N,), lambda: (0,)),
            pl.BlockSpec((N, N), lambda: (0, 0)),
        ],
        out_specs=[
            pl.BlockSpec((2, N), lambda: (0, 0)),
            pl.BlockSpec(memory_space=pltpu.SMEM),
        ],
        out_shape=[
            jax.ShapeDtypeStruct((2, N), jnp.float32),
            jax.ShapeDtypeStruct((2,), jnp.float32),
        ],
    )(P_1_AB, P_2_1_AB, P_1_BA, P_2_1_BA)

    partials = _make_sc_kernel()(
        samples.reshape(B * M * N), P_1_AB, cn[0], P_1_BA, cn[1],
        P_2_1_AB.reshape(-1), P_2_1_BA.reshape(-1))
    sums = jnp.sum(partials, axis=(0, 2))

    log_w = gamma - jax.scipy.special.logsumexp(gamma)
    m_ab = log_w[0] + sums[0] - B * nrm[0]
    m_ba = log_w[1] + sums[1] - B * nrm[1]
    return jnp.logaddexp(m_ab, m_ba)
